# baseline (speedup 1.0000x reference)
import functools
import jax
import jax.numpy as jnp
from jax import lax
from jax.experimental import pallas as pl
from jax.experimental.pallas import tpu as pltpu
from jax.experimental.pallas import tpu_sc as plsc

B, A, N = 1024, 32, 32
NW = 32
BPW = B // NW
L = 16


def _sc_body(lut_hbm, ptk_hbm, s_hbm, m_hbm, out_hbm,
             lut_v, ptk_v, s_v, m_v, y_v, res_v):
    wid = lax.axis_index("s") * 2 + lax.axis_index("c")
    base = wid * BPW

    pltpu.sync_copy(lut_hbm, lut_v)
    pltpu.sync_copy(ptk_hbm, ptk_v)
    pltpu.sync_copy(s_hbm.at[pl.ds(base * N, BPW * N)], s_v)
    pltpu.sync_copy(m_hbm.at[pl.ds(base * A, BPW * A)], m_v)

    lanes = lax.iota(jnp.int32, L)

    def b_loop(bl, _):
        srow = bl * N

        def g_loop(g, _):
            def k_loop(k, acc):
                pvec = ptk_v[pl.ds(k * (A * N) + g * L, L)]
                sk = plsc.load_gather(s_v, [jnp.full((L,), srow + k,
                                                     jnp.int32)])
                val = plsc.load_gather(lut_v, [(sk << 8) + pvec])
                return acc ^ val
            acc = lax.fori_loop(0, N, k_loop,
                                jnp.zeros((L,), jnp.int32))
            y_v[pl.ds(g * L, L)] = acc
            return 0
        lax.fori_loop(0, (A * N) // L, g_loop, 0)

        def j_loop(j, accs):
            a0, a1 = accs
            sj = plsc.load_gather(s_v, [jnp.full((L,), srow + j, jnp.int32)])
            y0 = plsc.load_gather(y_v, [lanes * N + j])
            y1 = plsc.load_gather(y_v, [(lanes + L) * N + j])
            a0 = a0 ^ plsc.load_gather(lut_v, [(y0 << 8) + sj])
            a1 = a1 ^ plsc.load_gather(lut_v, [(y1 << 8) + sj])
            return (a0, a1)
        z = jnp.zeros((L,), jnp.int32)
        c0, c1 = lax.fori_loop(0, N, j_loop, (z, z))

        m0 = m_v[pl.ds(bl * A, L)]
        m1 = m_v[pl.ds(bl * A + L, L)]
        tot = jnp.sum((c0 - m0) + (c1 - m1))
        res = jnp.maximum(1 - tot, 0)
        plsc.store_scatter(res_v, [jnp.full((L,), bl, jnp.int32)],
                           jnp.full((L,), res, jnp.int32),
                           mask=lanes == 0)
        return 0

    lax.fori_loop(0, BPW, b_loop, 0)
    pltpu.sync_copy(res_v, out_hbm.at[pl.ds(base, BPW)])


def kernel(m, s, P, lookuptable):
    lut_flat = lookuptable.reshape(256 * 256)
    ptk = jnp.transpose(P, (2, 0, 1)).reshape(N * A * N)
    s_flat = s.reshape(B * N)
    m_flat = m.reshape(B * A)

    mesh = plsc.VectorSubcoreMesh(core_axis_name="c", subcore_axis_name="s")
    run = pl.kernel(
        _sc_body,
        mesh=mesh,
        compiler_params=pltpu.CompilerParams(needs_layout_passes=False),
        out_type=jax.ShapeDtypeStruct((B,), jnp.int32),
        scratch_types=[
            pltpu.VMEM((256 * 256,), jnp.int32),
            pltpu.VMEM((N * A * N,), jnp.int32),
            pltpu.VMEM((BPW * N,), jnp.int32),
            pltpu.VMEM((BPW * A,), jnp.int32),
            pltpu.VMEM((A * N,), jnp.int32),
            pltpu.VMEM((BPW,), jnp.int32),
        ],
    )
    return run(lut_flat, ptk, s_flat, m_flat)


# speedup vs baseline: 22.1889x; 22.1889x over previous
import jax
import jax.numpy as jnp
from jax.experimental import pallas as pl
from jax.experimental.pallas import tpu as pltpu

B, A, N = 1024, 32, 32


def _gf_kernel(s_ref, m_ref, p_ref, out_ref):
    s = s_ref[...]
    m = m_ref[...]
    pmat = p_ref[...]

    pt = jnp.concatenate(
        [jnp.swapaxes(pmat[:, j, :], 0, 1) for j in range(N)], axis=1)

    rowv = jax.lax.broadcasted_iota(jnp.int32, (N, N * A), 0)
    colv = jax.lax.broadcasted_iota(jnp.int32, (N, N * A), 1)
    rmat = jnp.where((colv >> 5) == rowv, 1, 0).astype(jnp.bfloat16)
    s_rep = jnp.dot(s.astype(jnp.bfloat16), rmat,
                    preferred_element_type=jnp.float32).astype(jnp.int32)

    s_bits = jnp.concatenate(
        [((s >> t) & 1).astype(jnp.bfloat16) for t in range(8)], axis=1)

    xt = pt
    blocks = []
    for t in range(8):
        blocks.append(xt)
        if t < 7:
            xt = ((xt << 1) & 0xFF) ^ (0x1D & (-((xt >> 7) & 1)))
    w_cols = []
    for u in range(8):
        w_cols.append(jnp.concatenate(
            [((bt >> u) & 1).astype(jnp.bfloat16) for bt in blocks], axis=0))
    w = jnp.concatenate(w_cols, axis=1)

    counts = jnp.dot(s_bits, w,
                     preferred_element_type=jnp.float32)
    bits = counts.astype(jnp.int32)

    zs = []
    for v in range(8):
        t = s_rep * (bits[:, v * N * A:(v + 1) * N * A] & 1)
        width = N * A
        while width > A:
            width //= 2
            t = t[:, :width] ^ t[:, width:2 * width]
        zs.append(t)
    acc = zs[7]
    for v in range(6, -1, -1):
        acc = zs[v] ^ (((acc << 1) & 0xFF) ^ (0x1D & (-((acc >> 7) & 1))))

    m_check = acc - m
    out_ref[...] = jnp.maximum(1 - jnp.sum(m_check, axis=1, keepdims=True), 0)


def kernel(m, s, P, lookuptable):
    del lookuptable
    bc = B
    out = pl.pallas_call(
        _gf_kernel,
        grid=(B // bc,),
        in_specs=[
            pl.BlockSpec((bc, N), lambda i: (i, 0)),
            pl.BlockSpec((bc, A), lambda i: (i, 0)),
            pl.BlockSpec((A, N, N), lambda i: (0, 0, 0)),
        ],
        out_specs=pl.BlockSpec((bc, 1), lambda i: (i, 0)),
        out_shape=jax.ShapeDtypeStruct((B, 1), jnp.int32),
    )(s, m, P)
    return out.reshape(B)


# speedup vs baseline: 22.6796x; 1.0221x over previous
import jax
import jax.numpy as jnp
from jax.experimental import pallas as pl
from jax.experimental.pallas import tpu as pltpu

B, A, N = 1024, 32, 32


def _gf_kernel(s_ref, m_ref, p_ref, out_ref):
    s = s_ref[...]
    m = m_ref[...]
    pmat = p_ref[...]

    pt = jnp.concatenate(
        [jnp.swapaxes(pmat[:, j, :], 0, 1) for j in range(N)], axis=1)

    rowv = jax.lax.broadcasted_iota(jnp.int32, (N, N * A), 0)
    colv = jax.lax.broadcasted_iota(jnp.int32, (N, N * A), 1)
    rmat = jnp.where((colv >> 5) == rowv, 1, 0).astype(jnp.bfloat16)
    s_rep = jnp.dot(s.astype(jnp.bfloat16), rmat,
                    preferred_element_type=jnp.float32).astype(jnp.int32)

    s_bits = jnp.concatenate(
        [((s >> t) & 1).astype(jnp.bfloat16) for t in range(8)], axis=1)

    xt = pt
    blocks = []
    for t in range(8):
        blocks.append(xt)
        if t < 7:
            xt = ((xt << 1) & 0xFF) ^ (0x1D & (-((xt >> 7) & 1)))
    w_cols = []
    for u in range(8):
        w_cols.append(jnp.concatenate(
            [((bt >> u) & 1).astype(jnp.bfloat16) for bt in blocks], axis=0))
    w = jnp.concatenate(w_cols, axis=1)

    counts = jnp.dot(s_bits, w,
                     preferred_element_type=jnp.float32)
    bits = jax.lax.bitcast_convert_type(counts + jnp.float32(8388608.0),
                                        jnp.int32)

    zs = []
    for v in range(8):
        t = s_rep * (bits[:, v * N * A:(v + 1) * N * A] & 1)
        width = N * A
        while width > A:
            width //= 2
            t = t[:, :width] ^ t[:, width:2 * width]
        zs.append(t)
    acc = zs[7]
    for v in range(6, -1, -1):
        acc = zs[v] ^ (((acc << 1) & 0xFF) ^ (0x1D & (-((acc >> 7) & 1))))

    m_check = acc - m
    out_ref[...] = jnp.maximum(1 - jnp.sum(m_check, axis=1, keepdims=True), 0)


def kernel(m, s, P, lookuptable):
    del lookuptable
    bc = B
    out = pl.pallas_call(
        _gf_kernel,
        grid=(B // bc,),
        in_specs=[
            pl.BlockSpec((bc, N), lambda i: (i, 0)),
            pl.BlockSpec((bc, A), lambda i: (i, 0)),
            pl.BlockSpec((A, N, N), lambda i: (0, 0, 0)),
        ],
        out_specs=pl.BlockSpec((bc, 1), lambda i: (i, 0)),
        out_shape=jax.ShapeDtypeStruct((B, 1), jnp.int32),
    )(s, m, P)
    return out.reshape(B)


# speedup vs baseline: 23.5540x; 1.0386x over previous
import jax
import jax.numpy as jnp
from jax.experimental import pallas as pl
from jax.experimental.pallas import tpu as pltpu

B, A, N = 1024, 32, 32


def _gf_kernel(s_ref, m_ref, p_ref, out_ref):
    s = s_ref[...]
    m = m_ref[...]
    pmat = p_ref[...]

    pt = jnp.concatenate(
        [jnp.swapaxes(pmat[:, j, :], 0, 1) for j in range(N)], axis=1)

    rowv = jax.lax.broadcasted_iota(jnp.int32, (N, N * A), 0)
    colv = jax.lax.broadcasted_iota(jnp.int32, (N, N * A), 1)
    rmat = jnp.where((colv >> 5) == rowv, 1, 0).astype(jnp.bfloat16)
    s_rep = jnp.dot(s.astype(jnp.bfloat16), rmat,
                    preferred_element_type=jnp.float32).astype(jnp.int32)

    s_bits = jnp.concatenate(
        [((s >> t) & 1).astype(jnp.bfloat16) for t in range(8)], axis=1)

    xt = pt
    blocks = []
    for t in range(8):
        blocks.append(xt)
        if t < 7:
            xt = ((xt << 1) & 0xFF) ^ (0x1D & (-((xt >> 7) & 1)))
    w_cols = []
    for u in range(8):
        w_cols.append(jnp.concatenate(
            [((bt >> u) & 1).astype(jnp.bfloat16) for bt in blocks], axis=0))
    w = jnp.concatenate(w_cols, axis=1)

    counts = jnp.dot(s_bits, w,
                     preferred_element_type=jnp.float32)
    bits = jax.lax.bitcast_convert_type(counts + jnp.float32(8388608.0),
                                        jnp.int32)

    zs = []
    for v in range(8):
        base = v * N * A
        acc = None
        for g in range(8):
            part = s_rep[:, g * 128:(g + 1) * 128] * \
                (bits[:, base + g * 128:base + (g + 1) * 128] & 1)
            acc = part if acc is None else acc ^ part
        acc = acc[:, :64] ^ acc[:, 64:]
        acc = acc[:, :32] ^ acc[:, 32:]
        zs.append(acc)
    acc = zs[7]
    for v in range(6, -1, -1):
        acc = zs[v] ^ (((acc << 1) & 0xFF) ^ (0x1D & (-((acc >> 7) & 1))))

    m_check = acc - m
    out_ref[...] = jnp.maximum(1 - jnp.sum(m_check, axis=1, keepdims=True), 0)


def kernel(m, s, P, lookuptable):
    del lookuptable
    bc = B
    out = pl.pallas_call(
        _gf_kernel,
        grid=(B // bc,),
        in_specs=[
            pl.BlockSpec((bc, N), lambda i: (i, 0)),
            pl.BlockSpec((bc, A), lambda i: (i, 0)),
            pl.BlockSpec((A, N, N), lambda i: (0, 0, 0)),
        ],
        out_specs=pl.BlockSpec((bc, 1), lambda i: (i, 0)),
        out_shape=jax.ShapeDtypeStruct((B, 1), jnp.int32),
    )(s, m, P)
    return out.reshape(B)
